# baseline (speedup 1.0000x reference)
import functools

import jax
import jax.numpy as jnp
from jax import lax
from jax.experimental import pallas as pl
from jax.experimental.pallas import tpu as pltpu
from jax.experimental.pallas import tpu_sc as plsc

B = 16
SEQ = 2048
TOTAL = B * SEQ
H = 1024
V = 100000
L = 16

ROWS_PER_BLOCK = 2048


NCHUNK = 2
CHUNK_B = B // NCHUNK
CHUNK_ROWS = TOTAL // NCHUNK


def _tw_body(hs_ref, w_ref, b_ref, out_ref):
    acc = jnp.dot(hs_ref[...], w_ref[...], preferred_element_type=jnp.float32)
    out_ref[...] = jnp.maximum(acc + b_ref[0, 0], 0.0)


def _token_weights_chunk(hidden_states, W, b, chunk):
    base_blk = chunk * (CHUNK_ROWS // ROWS_PER_BLOCK)
    return pl.pallas_call(
        _tw_body,
        grid=(CHUNK_ROWS // ROWS_PER_BLOCK,),
        in_specs=[
            pl.BlockSpec((ROWS_PER_BLOCK, H), lambda i: (i + base_blk, 0)),
            pl.BlockSpec((H, 1), lambda i: (0, 0)),
            pl.BlockSpec(memory_space=pltpu.SMEM),
        ],
        out_specs=pl.BlockSpec((ROWS_PER_BLOCK, 1), lambda i: (i, 0)),
        out_shape=jax.ShapeDtypeStruct((CHUNK_ROWS, 1), jnp.float32),
    )(hidden_states, W, b.reshape(1, 1))


HALF_V = V // 2


def _sc_scatter_body(ids_hbm, tw_hbm, out_ref, table_v, ids_v, tw_v, *, base_b):
    cid = lax.axis_index("c")
    sid = lax.axis_index("s")
    wid = sid * 2 + cid

    @pl.when(wid < 2 * CHUNK_B)
    def _():
        batch = base_b + wid // 2
        lo = (wid % 2) * HALF_V

        zeros = jnp.zeros((L,), jnp.float32)

        def zero_body(j, _):
            table_v[pl.ds(j * L, L)] = zeros
            return ()

        lax.fori_loop(0, HALF_V // L, zero_body, (), unroll=8)

        pltpu.sync_copy(ids_hbm.at[pl.ds(batch * SEQ, SEQ)], ids_v)
        pltpu.sync_copy(tw_hbm.at[pl.ds((batch - base_b) * SEQ, SEQ)], tw_v)

        def tok_body(j, _):
            idx = ids_v[pl.ds(j * L, L)] - lo
            w = tw_v[pl.ds(j * L, L)]
            in_r = (idx >= 0) & (idx < HALF_V)
            idx_c = jnp.clip(idx, 0, HALF_V - 1)
            cur = plsc.load_gather(table_v, [idx_c])

            def cond(cur):
                return jnp.any(in_r & (w > cur))

            def body(cur):
                plsc.store_scatter(table_v, [idx_c], w, mask=in_r & (w > cur))
                return plsc.load_gather(table_v, [idx_c])

            lax.while_loop(cond, body, cur)
            return ()

        lax.fori_loop(0, SEQ // L, tok_body, ())

        pltpu.sync_copy(table_v, out_ref.at[pl.ds(batch * V + lo, HALF_V)])


def _make_sc_scatter(base_b):
    return functools.partial(
        pl.kernel,
        out_type=(),
        mesh=plsc.VectorSubcoreMesh(core_axis_name="c", subcore_axis_name="s"),
        compiler_params=pltpu.CompilerParams(needs_layout_passes=False),
        scratch_types=[
            pltpu.VMEM((HALF_V,), jnp.float32),
            pltpu.VMEM((SEQ,), jnp.int32),
            pltpu.VMEM((SEQ,), jnp.float32),
        ],
    )(functools.partial(_sc_scatter_body, base_b=base_b))


_sc_scatters = [_make_sc_scatter(c * CHUNK_B) for c in range(NCHUNK)]


@jax.jit
def kernel(hidden_states, extend_seq_lens, input_ids, W, b):
    del extend_seq_lens
    ids = input_ids.astype(jnp.int32)
    out_ref = jax.new_ref(jnp.zeros((B * V,), jnp.float32))
    for c in range(NCHUNK):
        tw = _token_weights_chunk(hidden_states, W, b, c).reshape(CHUNK_ROWS)
        _sc_scatters[c](ids, tw, out_ref)
    return out_ref[...].reshape(B, V)


# speedup vs baseline: 1.0880x; 1.0880x over previous
import functools

import jax
import jax.numpy as jnp
from jax import lax
from jax.experimental import pallas as pl
from jax.experimental.pallas import tpu as pltpu
from jax.experimental.pallas import tpu_sc as plsc

B = 16
SEQ = 2048
TOTAL = B * SEQ
H = 1024
V = 100000
L = 16

ROWS_PER_BLOCK = 2048
NSTREAM = 2
HALF_ROWS = TOTAL // NSTREAM


def _tw_body(hs_a_ref, hs_b_ref, w_ref, b_ref, out_ref):
    wa = jnp.dot(hs_a_ref[...], w_ref[...], preferred_element_type=jnp.float32)
    wb = jnp.dot(hs_b_ref[...], w_ref[...], preferred_element_type=jnp.float32)
    acc = jnp.stack([wa, wb], axis=0)
    out_ref[...] = jnp.maximum(acc + b_ref[0, 0], 0.0)


def _token_weights(hidden_states, W, b):
    nblk = HALF_ROWS // ROWS_PER_BLOCK
    out = pl.pallas_call(
        _tw_body,
        grid=(nblk,),
        in_specs=[
            pl.BlockSpec((ROWS_PER_BLOCK, H), lambda i: (i, 0)),
            pl.BlockSpec((ROWS_PER_BLOCK, H), lambda i: (i + nblk, 0)),
            pl.BlockSpec((H, 1), lambda i: (0, 0)),
            pl.BlockSpec(memory_space=pltpu.SMEM),
        ],
        out_specs=pl.BlockSpec((NSTREAM, ROWS_PER_BLOCK, 1), lambda i: (0, i, 0)),
        out_shape=jax.ShapeDtypeStruct((NSTREAM, HALF_ROWS, 1), jnp.float32),
    )(hidden_states, hidden_states, W, b.reshape(1, 1))
    return out.reshape(TOTAL)


HALF_V = V // 2


def _sc_scatter_body(ids_hbm, tw_hbm, out_hbm, table_v, ids_v, tw_v):
    cid = lax.axis_index("c")
    sid = lax.axis_index("s")
    wid = sid * 2 + cid
    batch = wid // 2
    lo = (wid % 2) * HALF_V

    zeros = jnp.zeros((L,), jnp.float32)

    def zero_body(j, _):
        table_v[pl.ds(j * L, L)] = zeros
        return ()

    lax.fori_loop(0, HALF_V // L, zero_body, (), unroll=8)

    pltpu.sync_copy(ids_hbm.at[pl.ds(batch * SEQ, SEQ)], ids_v)
    pltpu.sync_copy(tw_hbm.at[pl.ds(batch * SEQ, SEQ)], tw_v)

    def tok_body(j, _):
        idx = ids_v[pl.ds(j * L, L)] - lo
        w = tw_v[pl.ds(j * L, L)]
        in_r = (idx >= 0) & (idx < HALF_V)
        idx_c = jnp.clip(idx, 0, HALF_V - 1)
        cur = plsc.load_gather(table_v, [idx_c])

        def cond(cur):
            return jnp.any(in_r & (w > cur))

        def body(cur):
            plsc.store_scatter(table_v, [idx_c], w, mask=in_r & (w > cur))
            return plsc.load_gather(table_v, [idx_c])

        lax.while_loop(cond, body, cur)
        return ()

    lax.fori_loop(0, SEQ // L, tok_body, ())

    pltpu.sync_copy(table_v, out_hbm.at[pl.ds(batch * V + lo, HALF_V)])


_sc_scatter = functools.partial(
    pl.kernel,
    out_type=jax.ShapeDtypeStruct((B * V,), jnp.float32),
    mesh=plsc.VectorSubcoreMesh(core_axis_name="c", subcore_axis_name="s"),
    compiler_params=pltpu.CompilerParams(needs_layout_passes=False),
    scratch_types=[
        pltpu.VMEM((HALF_V,), jnp.float32),
        pltpu.VMEM((SEQ,), jnp.int32),
        pltpu.VMEM((SEQ,), jnp.float32),
    ],
)(_sc_scatter_body)


@jax.jit
def kernel(hidden_states, extend_seq_lens, input_ids, W, b):
    del extend_seq_lens
    tw = _token_weights(hidden_states, W, b)
    ids = input_ids.astype(jnp.int32)
    flat = _sc_scatter(ids, tw)
    return flat.reshape(B, V)
